# baseline (speedup 1.0000x reference)
import functools

import jax
import jax.numpy as jnp
from jax import lax
from jax.experimental import pallas as pl
from jax.experimental.pallas import tpu as pltpu
from jax.experimental.pallas import tpu_sc as plsc

NC = 2
NS = 16
L = 16
NW = NC * NS

N_TOKENS = 204800
N_SLOTS = 128
D = 64

B_PER_W = N_TOKENS // NW
CHUNK = 128
N_CHUNKS = B_PER_W // CHUNK


def _sc_body(ids_hbm, first_hbm, emb_hbm, out_hbm,
             ids_v, idx_v, first_v, rows_v, sem):
    wid = lax.axis_index("s") * NC + lax.axis_index("c")
    base = wid * B_PER_W

    pltpu.sync_copy(first_hbm.at[pl.ds(0, L)], first_v)
    lo = first_v[...] - lax.iota(jnp.int32, L)
    hi = lo + N_SLOTS

    pltpu.sync_copy(ids_hbm.at[pl.ds(base, B_PER_W)], ids_v)

    def chunk(c, carry):
        for j in range(CHUNK // L):
            t = ids_v[pl.ds(c * CHUNK + j * L, L)]
            ok = (t >= lo) & (t < hi)
            idx_v[c, pl.ds(j * L, L)] = jnp.where(ok, t - lo, 0)
        pltpu.async_copy(emb_hbm.at[idx_v.at[c]], rows_v, sem).wait()
        pltpu.sync_copy(rows_v, out_hbm.at[pl.ds(base + c * CHUNK, CHUNK)])
        return carry

    lax.fori_loop(0, N_CHUNKS, chunk, 0)


@jax.jit
def _lookup(ids, input_ids, emb_weight):
    mesh = plsc.VectorSubcoreMesh(core_axis_name="c", subcore_axis_name="s",
                                  num_cores=NC, num_subcores=NS)
    f = pl.kernel(
        _sc_body,
        out_type=jax.ShapeDtypeStruct((N_TOKENS, D), jnp.float32),
        mesh=mesh,
        scratch_types=[
            pltpu.VMEM((B_PER_W,), jnp.int32),
            pltpu.VMEM((N_CHUNKS, CHUNK), jnp.int32),
            pltpu.VMEM((L,), jnp.int32),
            pltpu.VMEM((CHUNK, D), jnp.float32),
            pltpu.SemaphoreType.DMA,
        ],
        compiler_params=pltpu.CompilerParams(use_tc_tiling_on_sc=False),
    )
    return f(ids, input_ids, emb_weight)


def kernel(prompt_token_ids, input_ids, emb_weight):
    ids = prompt_token_ids.reshape(-1)
    return _lookup(ids, input_ids, emb_weight)


# speedup vs baseline: 1.0018x; 1.0018x over previous
import functools

import jax
import jax.numpy as jnp
from jax import lax
from jax.experimental import pallas as pl
from jax.experimental.pallas import tpu as pltpu
from jax.experimental.pallas import tpu_sc as plsc

NC = 2
NS = 16
L = 16
NW = NC * NS

N_TOKENS = 204800
N_SLOTS = 128
D = 64

B_PER_W = N_TOKENS // NW
CHUNK = 128
N_CHUNKS = B_PER_W // CHUNK


G_PER_BLK = 5
BLK = CHUNK * G_PER_BLK
N_BLK = B_PER_W // BLK
NBUF = 2


def _sc_body(ids_hbm, first_hbm, emb_hbm, out_hbm,
             ids_v, idx_v, first_v, rows_v, gsem, ssem):
    wid = lax.axis_index("s") * NC + lax.axis_index("c")
    base = wid * B_PER_W

    pltpu.sync_copy(first_hbm.at[pl.ds(0, L)], first_v)
    lo = first_v[...] - lax.iota(jnp.int32, L)
    hi = lo + N_SLOTS

    pltpu.sync_copy(ids_hbm.at[pl.ds(base, B_PER_W)], ids_v)

    def remap(c, carry):
        for j in range(CHUNK // L):
            t = ids_v[pl.ds(c * CHUNK + j * L, L)]
            ok = (t >= lo) & (t < hi)
            idx_v[c, pl.ds(j * L, L)] = jnp.where(ok, t - lo, 0)
        return carry

    lax.fori_loop(0, N_CHUNKS, remap, 0)

    def fire_block(blk):
        b = blk % NBUF
        for s in range(G_PER_BLK):
            g = blk * G_PER_BLK + s
            pltpu.async_copy(emb_hbm.at[idx_v.at[g]],
                             rows_v.at[b, pl.ds(s * CHUNK, CHUNK)], gsem)

    def store_handle(blk):
        b = blk % NBUF
        return pltpu.make_async_copy(
            rows_v.at[b], out_hbm.at[pl.ds(base + blk * BLK, BLK)], ssem)

    fire_block(0)
    for blk in range(N_BLK):
        if blk + 1 < N_BLK:
            fire_block(blk + 1)
        for s in range(G_PER_BLK):
            g = blk * G_PER_BLK + s
            b = blk % NBUF
            pltpu.make_async_copy(
                emb_hbm.at[idx_v.at[g]],
                rows_v.at[b, pl.ds(s * CHUNK, CHUNK)], gsem).wait()
        if blk >= NBUF:
            store_handle(blk - NBUF).wait()
        store_handle(blk).start()
    for blk in range(N_BLK - NBUF, N_BLK):
        store_handle(blk).wait()


@jax.jit
def _lookup(ids, input_ids, emb_weight):
    mesh = plsc.VectorSubcoreMesh(core_axis_name="c", subcore_axis_name="s",
                                  num_cores=NC, num_subcores=NS)
    f = pl.kernel(
        _sc_body,
        out_type=jax.ShapeDtypeStruct((N_TOKENS, D), jnp.float32),
        mesh=mesh,
        scratch_types=[
            pltpu.VMEM((B_PER_W,), jnp.int32),
            pltpu.VMEM((N_CHUNKS, CHUNK), jnp.int32),
            pltpu.VMEM((L,), jnp.int32),
            pltpu.VMEM((NBUF, BLK, D), jnp.float32),
            pltpu.SemaphoreType.DMA,
            pltpu.SemaphoreType.DMA,
        ],
        compiler_params=pltpu.CompilerParams(use_tc_tiling_on_sc=False),
    )
    return f(ids, input_ids, emb_weight)


def kernel(prompt_token_ids, input_ids, emb_weight):
    ids = prompt_token_ids.reshape(-1)
    return _lookup(ids, input_ids, emb_weight)


# speedup vs baseline: 23.8487x; 23.8050x over previous
import functools

import jax
import jax.numpy as jnp
from jax import lax
from jax.experimental import pallas as pl
from jax.experimental.pallas import tpu as pltpu
from jax.experimental.pallas import tpu_sc as plsc

NC = 2
NS = 16
L = 16
NW = NC * NS

N_TOKENS = 204800
N_SLOTS = 128
D = 64

B_PER_W = N_TOKENS // NW
CHUNK = 128
N_CHUNKS = B_PER_W // CHUNK


G_PER_BLK = 5
BLK = CHUNK * G_PER_BLK
N_BLK = B_PER_W // BLK
NBUF = 2


N_EXT = 2 * N_SLOTS


def _sc_body(ids_hbm, first_hbm, emb_hbm, out_hbm,
             ids_v, idx_v, first_v, bld_v, rows_v, shared, gsem, ssem):
    sid = lax.axis_index("s")
    wid = sid * NC + lax.axis_index("c")
    base = wid * B_PER_W

    pltpu.sync_copy(first_hbm.at[pl.ds(0, L)], first_v)
    lo = first_v[...] - lax.iota(jnp.int32, L)
    hi = lo + N_SLOTS

    @pl.when(sid == 0)
    def _build():
        pltpu.sync_copy(emb_hbm, bld_v.at[pl.ds(0, N_SLOTS)])
        r0 = [bld_v[0, pl.ds(k * L, L)] for k in range(D // L)]

        def rep(r, carry):
            for k in range(D // L):
                bld_v[N_SLOTS + r, pl.ds(k * L, L)] = r0[k]
            return carry

        lax.fori_loop(0, N_SLOTS, rep, 0)
        pltpu.sync_copy(bld_v, shared)

    pltpu.sync_copy(ids_hbm.at[pl.ds(base, B_PER_W)], ids_v)

    def remap(c, carry):
        for j in range(CHUNK // L):
            t = ids_v[pl.ds(c * CHUNK + j * L, L)]
            ok = (t >= lo) & (t < hi)
            idx_v[c, pl.ds(j * L, L)] = jnp.where(
                ok, t - lo, N_SLOTS + (t & (N_SLOTS - 1)))
        return carry

    lax.fori_loop(0, N_CHUNKS, remap, 0)
    plsc.subcore_barrier()

    def fire_block(blk):
        b = blk % NBUF
        for s in range(G_PER_BLK):
            g = blk * G_PER_BLK + s
            pltpu.async_copy(shared.at[idx_v.at[g]],
                             rows_v.at[b, pl.ds(s * CHUNK, CHUNK)], gsem)

    def store_handle(blk):
        b = blk % NBUF
        return pltpu.make_async_copy(
            rows_v.at[b], out_hbm.at[pl.ds(base + blk * BLK, BLK)], ssem)

    fire_block(0)
    for blk in range(N_BLK):
        if blk + 1 < N_BLK:
            fire_block(blk + 1)
        for s in range(G_PER_BLK):
            g = blk * G_PER_BLK + s
            b = blk % NBUF
            pltpu.make_async_copy(
                shared.at[idx_v.at[g]],
                rows_v.at[b, pl.ds(s * CHUNK, CHUNK)], gsem).wait()
        if blk >= NBUF:
            store_handle(blk - NBUF).wait()
        store_handle(blk).start()
    for blk in range(N_BLK - NBUF, N_BLK):
        store_handle(blk).wait()


@jax.jit
def _lookup(ids, input_ids, emb_weight):
    mesh = plsc.VectorSubcoreMesh(core_axis_name="c", subcore_axis_name="s",
                                  num_cores=NC, num_subcores=NS)
    f = pl.kernel(
        _sc_body,
        out_type=jax.ShapeDtypeStruct((N_TOKENS, D), jnp.float32),
        mesh=mesh,
        scratch_types=[
            pltpu.VMEM((B_PER_W,), jnp.int32),
            pltpu.VMEM((N_CHUNKS, CHUNK), jnp.int32),
            pltpu.VMEM((L,), jnp.int32),
            pltpu.VMEM((N_EXT, D), jnp.float32),
            pltpu.VMEM((NBUF, BLK, D), jnp.float32),
            pltpu.VMEM_SHARED((N_EXT, D), jnp.float32),
            pltpu.SemaphoreType.DMA,
            pltpu.SemaphoreType.DMA,
        ],
        compiler_params=pltpu.CompilerParams(use_tc_tiling_on_sc=False),
    )
    return f(ids, input_ids, emb_weight)


def kernel(prompt_token_ids, input_ids, emb_weight):
    ids = prompt_token_ids.reshape(-1)
    return _lookup(ids, input_ids, emb_weight)
